# baseline (speedup 1.0000x reference)
import functools

import jax
import jax.numpy as jnp
from jax import lax
from jax.experimental import pallas as pl
from jax.experimental.pallas import tpu as pltpu
from jax.experimental.pallas import tpu_sc as plsc

VOCAB = 1000000
EMBED = 64
HIDDEN = 128
NOUT = 2
BATCH = 4096
SEQ = 200

NC = 2
NS = 16
NW = NC * NS
B_PER_W = BATCH // NW
CHUNK = 100
CHUNKS_PER_ITEM = SEQ // CHUNK
N_CHUNKS = B_PER_W * CHUNKS_PER_ITEM
NBUF = 4
N_OUTER = N_CHUNKS // NBUF
NLANE = 16
NVEC = EMBED // NLANE
HSPLIT = 512000


def _sc_gather_sum(xh, hoff, table2):
    mesh = plsc.VectorSubcoreMesh(core_axis_name="c", subcore_axis_name="s")

    @functools.partial(
        pl.kernel,
        mesh=mesh,
        out_type=jax.ShapeDtypeStruct((BATCH, EMBED), jnp.float32),
        compiler_params=pltpu.CompilerParams(use_tc_tiling_on_sc=False),
        scratch_types=(
            [pltpu.VMEM((N_CHUNKS, CHUNK), jnp.int32),
             pltpu.VMEM((B_PER_W, EMBED), jnp.float32)]
            + [pltpu.VMEM((CHUNK, 128), jnp.float32) for _ in range(NBUF)]
            + [pltpu.VMEM((CHUNK + NLANE,), jnp.int32) for _ in range(NBUF)]
            + [pltpu.SemaphoreType.DMA for _ in range(2 * NBUF)]
        ),
    )
    def k(xh_hbm, hoff_hbm, table_hbm, out_hbm, idx_v, out_v, *rest):
        bufs = rest[:NBUF]
        hbufs = rest[NBUF:2 * NBUF]
        sems = rest[2 * NBUF:3 * NBUF]
        hsems = rest[3 * NBUF:]
        lanes = lax.iota(jnp.int32, NLANE)
        wid = lax.axis_index("s") * NC + lax.axis_index("c")
        ibase = wid * N_CHUNKS
        obase = wid * B_PER_W

        pltpu.sync_copy(xh_hbm.at[pl.ds(ibase, N_CHUNKS)], idx_v)

        def fire(chunk, b):
            pltpu.async_copy(table_hbm.at[idx_v.at[chunk]], bufs[b], sems[b])
            pltpu.async_copy(hoff_hbm.at[ibase + chunk],
                             hbufs[b].at[pl.ds(0, CHUNK)], hsems[b])

        def wait(chunk, b):
            pltpu.make_async_copy(
                table_hbm.at[idx_v.at[chunk]], bufs[b], sems[b]).wait()
            pltpu.make_async_copy(
                hoff_hbm.at[ibase + chunk],
                hbufs[b].at[pl.ds(0, CHUNK)], hsems[b]).wait()

        for b in range(NBUF):
            fire(b, b)

        def accum(buf, hs, accs):
            def body(s, a):
                off = hs[pl.ds(s, NLANE)][0]
                return tuple(
                    a[c] + buf[s, pl.ds(off + c * NLANE, NLANE)]
                    for c in range(NVEC)
                )
            return lax.fori_loop(0, CHUNK, body, accs)

        def outer(t, carry):
            for pair in range(NBUF // CHUNKS_PER_ITEM):
                accs = tuple(
                    jnp.zeros((NLANE,), jnp.float32) for _ in range(NVEC)
                )
                for half in range(CHUNKS_PER_ITEM):
                    b = pair * CHUNKS_PER_ITEM + half
                    chunk = t * NBUF + b
                    wait(chunk, b)
                    accs = accum(bufs[b], hbufs[b], accs)

                    @pl.when(t < N_OUTER - 1)
                    def _fire():
                        fire(chunk + NBUF, b)

                item = t * (NBUF // CHUNKS_PER_ITEM) + pair
                for c in range(NVEC):
                    out_v[item, pl.ds(c * NLANE, NLANE)] = accs[c]
            return carry

        lax.fori_loop(0, N_OUTER, outer, 0)
        pltpu.sync_copy(out_v, out_hbm.at[pl.ds(obase, B_PER_W)])

    return k(xh, hoff, table2)


VBLK = 512
FMT_GRID = HSPLIT // VBLK
LAST_BLK = -(-VOCAB // VBLK) - 1


def _fmt_body(ta_ref, tb_ref, eye_ref, out_ref):
    e = eye_ref[...]
    dn = (((0,), (0,)), ((), ()))
    out_ref[:, :EMBED] = lax.dot_general(
        ta_ref[...], e, dn, precision=lax.Precision.HIGHEST,
        preferred_element_type=jnp.float32)
    out_ref[:, EMBED:] = lax.dot_general(
        tb_ref[...], e, dn, precision=lax.Precision.HIGHEST,
        preferred_element_type=jnp.float32)


def _tc_format(tableT, eye):
    return pl.pallas_call(
        _fmt_body,
        grid=(FMT_GRID,),
        in_specs=[
            pl.BlockSpec((EMBED, VBLK), lambda i: (0, i)),
            pl.BlockSpec(
                (EMBED, VBLK),
                lambda i: (0, jnp.minimum(i + FMT_GRID, LAST_BLK)),
            ),
            pl.BlockSpec((EMBED, EMBED), lambda i: (0, 0)),
        ],
        out_specs=pl.BlockSpec((VBLK, 2 * EMBED), lambda i: (i, 0)),
        out_shape=jax.ShapeDtypeStruct((HSPLIT, 2 * EMBED), jnp.float32),
    )(tableT, tableT, eye)


BM = 512
NOUT_PAD = 128


def _mlp_body(s_ref, w1_ref, b1_ref, w2_ref, b2_ref, o_ref):
    h = jnp.dot(s_ref[...] * (1.0 / SEQ), w1_ref[...],
                preferred_element_type=jnp.float32)
    h = jnp.maximum(h + b1_ref[...], 0.0)
    o_ref[...] = jnp.dot(h, w2_ref[...],
                         preferred_element_type=jnp.float32) + b2_ref[...]


def kernel(x, table, W1, b1, W2, b2):
    xi = x.astype(jnp.int32)
    hi = (xi >= HSPLIT).astype(jnp.int32)
    xh = (xi - hi * HSPLIT).reshape(BATCH * SEQ // CHUNK, CHUNK)
    hoff = (hi << 6).reshape(BATCH * SEQ // CHUNK, CHUNK)
    table2 = _tc_format(table.T, jnp.eye(EMBED, dtype=jnp.float32))
    sums = _sc_gather_sum(xh, hoff, table2)

    w2p = jnp.zeros((HIDDEN, NOUT_PAD), W2.dtype).at[:, :NOUT].set(W2)
    b2p = jnp.zeros((1, NOUT_PAD), b2.dtype).at[0, :NOUT].set(b2)
    b1r = b1.reshape(1, HIDDEN)

    out = pl.pallas_call(
        _mlp_body,
        grid=(BATCH // BM,),
        in_specs=[
            pl.BlockSpec((BM, EMBED), lambda i: (i, 0)),
            pl.BlockSpec((EMBED, HIDDEN), lambda i: (0, 0)),
            pl.BlockSpec((1, HIDDEN), lambda i: (0, 0)),
            pl.BlockSpec((HIDDEN, NOUT_PAD), lambda i: (0, 0)),
            pl.BlockSpec((1, NOUT_PAD), lambda i: (0, 0)),
        ],
        out_specs=pl.BlockSpec((BM, NOUT_PAD), lambda i: (i, 0)),
        out_shape=jax.ShapeDtypeStruct((BATCH, NOUT_PAD), jnp.float32),
    )(sums, W1, b1r, w2p, b2p)
    return out[:, :NOUT]


# speedup vs baseline: 1.2190x; 1.2190x over previous
import functools

import jax
import jax.numpy as jnp
from jax import lax
from jax.experimental import pallas as pl
from jax.experimental.pallas import tpu as pltpu
from jax.experimental.pallas import tpu_sc as plsc

VOCAB = 1000000
EMBED = 64
HIDDEN = 128
NOUT = 2
BATCH = 4096
SEQ = 200

NC = 2
NS = 16
NW = NC * NS
B_PER_W = BATCH // NW
CHUNK = 100
CHUNKS_PER_ITEM = SEQ // CHUNK
N_CHUNKS = B_PER_W * CHUNKS_PER_ITEM
NBUF = 4
N_OUTER = N_CHUNKS // NBUF
NLANE = 16
NVEC = EMBED // NLANE
HSPLIT = 512000


def _sc_gather_sum(xh, hoff, table2):
    mesh = plsc.VectorSubcoreMesh(core_axis_name="c", subcore_axis_name="s")

    @functools.partial(
        pl.kernel,
        mesh=mesh,
        out_type=jax.ShapeDtypeStruct((BATCH, EMBED), jnp.float32),
        compiler_params=pltpu.CompilerParams(use_tc_tiling_on_sc=False),
        scratch_types=(
            [pltpu.VMEM((N_CHUNKS, CHUNK), jnp.int32),
             pltpu.VMEM((B_PER_W, EMBED), jnp.float32)]
            + [pltpu.VMEM((CHUNK, 128), jnp.float32) for _ in range(NBUF)]
            + [pltpu.VMEM((CHUNK + NLANE,), jnp.int32) for _ in range(NBUF)]
            + [pltpu.SemaphoreType.DMA for _ in range(2 * NBUF)]
        ),
    )
    def k(xh_hbm, hoff_hbm, table_hbm, out_hbm, idx_v, out_v, *rest):
        bufs = rest[:NBUF]
        hbufs = rest[NBUF:2 * NBUF]
        sems = rest[2 * NBUF:3 * NBUF]
        hsems = rest[3 * NBUF:]
        lanes = lax.iota(jnp.int32, NLANE)
        wid = lax.axis_index("s") * NC + lax.axis_index("c")
        ibase = wid * N_CHUNKS
        obase = wid * B_PER_W

        pltpu.sync_copy(xh_hbm.at[pl.ds(ibase, N_CHUNKS)], idx_v)

        def fire(chunk, b):
            pltpu.async_copy(table_hbm.at[idx_v.at[chunk]], bufs[b], sems[b])
            pltpu.async_copy(hoff_hbm.at[ibase + chunk],
                             hbufs[b].at[pl.ds(0, CHUNK)], hsems[b])

        def wait(chunk, b):
            pltpu.make_async_copy(
                table_hbm.at[idx_v.at[chunk]], bufs[b], sems[b]).wait()
            pltpu.make_async_copy(
                hoff_hbm.at[ibase + chunk],
                hbufs[b].at[pl.ds(0, CHUNK)], hsems[b]).wait()

        for b in range(NBUF):
            fire(b, b)

        def accum(buf, hs, accs):
            def body(s, a):
                off = hs[pl.ds(s, NLANE)][0]
                return tuple(
                    a[c] + buf[s, pl.ds(off + c * NLANE, NLANE)]
                    for c in range(NVEC)
                )
            return lax.fori_loop(0, CHUNK, body, accs)

        def outer(t, carry):
            for pair in range(NBUF // CHUNKS_PER_ITEM):
                accs = tuple(
                    jnp.zeros((NLANE,), jnp.float32) for _ in range(NVEC)
                )
                for half in range(CHUNKS_PER_ITEM):
                    b = pair * CHUNKS_PER_ITEM + half
                    chunk = t * NBUF + b
                    wait(chunk, b)
                    accs = accum(bufs[b], hbufs[b], accs)

                    @pl.when(t < N_OUTER - 1)
                    def _fire():
                        fire(chunk + NBUF, b)

                item = t * (NBUF // CHUNKS_PER_ITEM) + pair
                for c in range(NVEC):
                    out_v[item, pl.ds(c * NLANE, NLANE)] = accs[c]
            return carry

        lax.fori_loop(0, N_OUTER, outer, 0)
        pltpu.sync_copy(out_v, out_hbm.at[pl.ds(obase, B_PER_W)])

    return k(xh, hoff, table2)


VBLK = 512
FMT_GRID = HSPLIT // VBLK
LAST_BLK = -(-VOCAB // VBLK) - 1


def _fmt_body(ta_ref, tb_ref, eye_ref, out_ref):
    del eye_ref
    out_ref[:, :EMBED] = ta_ref[...].T
    out_ref[:, EMBED:] = tb_ref[...].T


def _tc_format(tableT, eye):
    return pl.pallas_call(
        _fmt_body,
        grid=(FMT_GRID,),
        in_specs=[
            pl.BlockSpec((EMBED, VBLK), lambda i: (0, i)),
            pl.BlockSpec(
                (EMBED, VBLK),
                lambda i: (0, jnp.minimum(i + FMT_GRID, LAST_BLK)),
            ),
            pl.BlockSpec((EMBED, EMBED), lambda i: (0, 0)),
        ],
        out_specs=pl.BlockSpec((VBLK, 2 * EMBED), lambda i: (i, 0)),
        out_shape=jax.ShapeDtypeStruct((HSPLIT, 2 * EMBED), jnp.float32),
    )(tableT, tableT, eye)


BM = 512
NOUT_PAD = 128


def _mlp_body(s_ref, w1_ref, b1_ref, w2_ref, b2_ref, o_ref):
    h = jnp.dot(s_ref[...] * (1.0 / SEQ), w1_ref[...],
                preferred_element_type=jnp.float32)
    h = jnp.maximum(h + b1_ref[...], 0.0)
    o_ref[...] = jnp.dot(h, w2_ref[...],
                         preferred_element_type=jnp.float32) + b2_ref[...]


def kernel(x, table, W1, b1, W2, b2):
    xi = x.astype(jnp.int32)
    hi = (xi >= HSPLIT).astype(jnp.int32)
    xh = (xi - hi * HSPLIT).reshape(BATCH * SEQ // CHUNK, CHUNK)
    hoff = (hi << 6).reshape(BATCH * SEQ // CHUNK, CHUNK)
    table2 = _tc_format(table.T, jnp.eye(EMBED, dtype=jnp.float32))
    sums = _sc_gather_sum(xh, hoff, table2)

    w2p = jnp.zeros((HIDDEN, NOUT_PAD), W2.dtype).at[:, :NOUT].set(W2)
    b2p = jnp.zeros((1, NOUT_PAD), b2.dtype).at[0, :NOUT].set(b2)
    b1r = b1.reshape(1, HIDDEN)

    out = pl.pallas_call(
        _mlp_body,
        grid=(BATCH // BM,),
        in_specs=[
            pl.BlockSpec((BM, EMBED), lambda i: (i, 0)),
            pl.BlockSpec((EMBED, HIDDEN), lambda i: (0, 0)),
            pl.BlockSpec((1, HIDDEN), lambda i: (0, 0)),
            pl.BlockSpec((HIDDEN, NOUT_PAD), lambda i: (0, 0)),
            pl.BlockSpec((1, NOUT_PAD), lambda i: (0, 0)),
        ],
        out_specs=pl.BlockSpec((BM, NOUT_PAD), lambda i: (i, 0)),
        out_shape=jax.ShapeDtypeStruct((BATCH, NOUT_PAD), jnp.float32),
    )(sums, W1, b1r, w2p, b2p)
    return out[:, :NOUT]


# speedup vs baseline: 2.0674x; 1.6960x over previous
import functools

import jax
import jax.numpy as jnp
from jax import lax
from jax.experimental import pallas as pl
from jax.experimental.pallas import tpu as pltpu
from jax.experimental.pallas import tpu_sc as plsc

VOCAB = 1000000
EMBED = 64
HIDDEN = 128
NOUT = 2
BATCH = 4096
SEQ = 200

NC = 2
NS = 16
NW = NC * NS
B_PER_W = BATCH // NW
CHUNK = 100
CHUNKS_PER_ITEM = SEQ // CHUNK
N_CHUNKS = B_PER_W * CHUNKS_PER_ITEM
NBUF = 4
N_OUTER = N_CHUNKS // NBUF
NLANE = 16
NVEC = EMBED // NLANE
HSPLIT = 512000


def _sc_gather_sum(xh, hoff, table2):
    mesh = plsc.VectorSubcoreMesh(core_axis_name="c", subcore_axis_name="s")

    @functools.partial(
        pl.kernel,
        mesh=mesh,
        out_type=jax.ShapeDtypeStruct((BATCH, EMBED), jnp.float32),
        compiler_params=pltpu.CompilerParams(use_tc_tiling_on_sc=False),
        scratch_types=(
            [pltpu.VMEM((N_CHUNKS, CHUNK), jnp.int32),
             pltpu.VMEM((B_PER_W, EMBED), jnp.float32)]
            + [pltpu.VMEM((CHUNK, 128), jnp.float32) for _ in range(NBUF)]
            + [pltpu.VMEM((CHUNK + NLANE,), jnp.int32) for _ in range(NBUF)]
            + [pltpu.SemaphoreType.DMA for _ in range(2 * NBUF)]
        ),
    )
    def k(xh_hbm, hoff_hbm, table_hbm, out_hbm, idx_v, out_v, *rest):
        bufs = rest[:NBUF]
        hbufs = rest[NBUF:2 * NBUF]
        sems = rest[2 * NBUF:3 * NBUF]
        hsems = rest[3 * NBUF:]
        lanes = lax.iota(jnp.int32, NLANE)
        wid = lax.axis_index("s") * NC + lax.axis_index("c")
        ibase = wid * N_CHUNKS
        obase = wid * B_PER_W

        pltpu.sync_copy(xh_hbm.at[pl.ds(ibase, N_CHUNKS)], idx_v)

        def fire(chunk, b):
            pltpu.async_copy(table_hbm.at[idx_v.at[chunk]], bufs[b], sems[b])
            pltpu.async_copy(hoff_hbm.at[ibase + chunk],
                             hbufs[b].at[pl.ds(0, CHUNK)], hsems[b])

        def wait(chunk, b):
            pltpu.make_async_copy(
                table_hbm.at[idx_v.at[chunk]], bufs[b], sems[b]).wait()
            pltpu.make_async_copy(
                hoff_hbm.at[ibase + chunk],
                hbufs[b].at[pl.ds(0, CHUNK)], hsems[b]).wait()

        for b in range(NBUF):
            fire(b, b)

        def accum(buf, hs, accs):
            def body(s, a):
                off = hs[pl.ds(s, NLANE)][0]
                return tuple(
                    a[c] + buf[s, pl.ds(off + c * NLANE, NLANE)]
                    for c in range(NVEC)
                )
            return lax.fori_loop(0, CHUNK, body, accs)

        def outer(t, carry):
            for pair in range(NBUF // CHUNKS_PER_ITEM):
                accs = tuple(
                    jnp.zeros((NLANE,), jnp.float32) for _ in range(NVEC)
                )
                for half in range(CHUNKS_PER_ITEM):
                    b = pair * CHUNKS_PER_ITEM + half
                    chunk = t * NBUF + b
                    wait(chunk, b)
                    accs = accum(bufs[b], hbufs[b], accs)

                    @pl.when(t < N_OUTER - 1)
                    def _fire():
                        fire(chunk + NBUF, b)

                item = t * (NBUF // CHUNKS_PER_ITEM) + pair
                for c in range(NVEC):
                    out_v[item, pl.ds(c * NLANE, NLANE)] = accs[c]
            return carry

        lax.fori_loop(0, N_OUTER, outer, 0)
        pltpu.sync_copy(out_v, out_hbm.at[pl.ds(obase, B_PER_W)])

    return k(xh, hoff, table2)


VBLK = 2048
FMT_GRID = HSPLIT // VBLK
LAST_BLK = -(-VOCAB // VBLK) - 1


def _fmt_body(ta_ref, tb_ref, out_ref):
    for q in range(VBLK // 128):
        a = ta_ref[:, pl.ds(128 * q, 128)].T
        b = tb_ref[:, pl.ds(128 * q, 128)].T
        out_ref[pl.ds(128 * q, 128), :] = jnp.concatenate([a, b], axis=1)


def _tc_format(tableT):
    return pl.pallas_call(
        _fmt_body,
        grid=(FMT_GRID,),
        in_specs=[
            pl.BlockSpec((EMBED, VBLK), lambda i: (0, i)),
            pl.BlockSpec(
                (EMBED, VBLK),
                lambda i: (0, jnp.minimum(i + FMT_GRID, LAST_BLK)),
            ),
        ],
        out_specs=pl.BlockSpec((VBLK, 2 * EMBED), lambda i: (i, 0)),
        out_shape=jax.ShapeDtypeStruct((HSPLIT, 2 * EMBED), jnp.float32),
        compiler_params=pltpu.CompilerParams(
            dimension_semantics=("parallel",)),
    )(tableT, tableT)


BM = 512
NOUT_PAD = 128


def _mlp_body(s_ref, w1_ref, b1_ref, w2_ref, b2_ref, o_ref):
    h = jnp.dot(s_ref[...] * (1.0 / SEQ), w1_ref[...],
                preferred_element_type=jnp.float32)
    h = jnp.maximum(h + b1_ref[...], 0.0)
    o_ref[...] = jnp.dot(h, w2_ref[...],
                         preferred_element_type=jnp.float32) + b2_ref[...]


def kernel(x, table, W1, b1, W2, b2):
    xi = x.astype(jnp.int32)
    hi = (xi >= HSPLIT).astype(jnp.int32)
    xh = (xi - hi * HSPLIT).reshape(BATCH * SEQ // CHUNK, CHUNK)
    hoff = (hi << 6).reshape(BATCH * SEQ // CHUNK, CHUNK)
    table2 = _tc_format(table.T)
    sums = _sc_gather_sum(xh, hoff, table2)

    w2p = jnp.zeros((HIDDEN, NOUT_PAD), W2.dtype).at[:, :NOUT].set(W2)
    b2p = jnp.zeros((1, NOUT_PAD), b2.dtype).at[0, :NOUT].set(b2)
    b1r = b1.reshape(1, HIDDEN)

    out = pl.pallas_call(
        _mlp_body,
        grid=(BATCH // BM,),
        in_specs=[
            pl.BlockSpec((BM, EMBED), lambda i: (i, 0)),
            pl.BlockSpec((EMBED, HIDDEN), lambda i: (0, 0)),
            pl.BlockSpec((1, HIDDEN), lambda i: (0, 0)),
            pl.BlockSpec((HIDDEN, NOUT_PAD), lambda i: (0, 0)),
            pl.BlockSpec((1, NOUT_PAD), lambda i: (0, 0)),
        ],
        out_specs=pl.BlockSpec((BM, NOUT_PAD), lambda i: (i, 0)),
        out_shape=jax.ShapeDtypeStruct((BATCH, NOUT_PAD), jnp.float32),
    )(sums, W1, b1r, w2p, b2p)
    return out[:, :NOUT]


# speedup vs baseline: 2.3819x; 1.1521x over previous
import functools

import jax
import jax.numpy as jnp
from jax import lax
from jax.experimental import pallas as pl
from jax.experimental.pallas import tpu as pltpu
from jax.experimental.pallas import tpu_sc as plsc

VOCAB = 1000000
EMBED = 64
HIDDEN = 128
NOUT = 2
BATCH = 4096
SEQ = 200

NC = 2
NS = 16
NW = NC * NS
B_PER_W = BATCH // NW
CHUNK = 100
CHUNKS_PER_ITEM = SEQ // CHUNK
N_CHUNKS = B_PER_W * CHUNKS_PER_ITEM
NBUF = 4
N_OUTER = N_CHUNKS // NBUF
NLANE = 16
NVEC = EMBED // NLANE
HSPLIT = 512000


def _sc_gather_sum(xh, hoff, table2):
    mesh = plsc.VectorSubcoreMesh(core_axis_name="c", subcore_axis_name="s")

    @functools.partial(
        pl.kernel,
        mesh=mesh,
        out_type=jax.ShapeDtypeStruct((BATCH, EMBED), jnp.float32),
        compiler_params=pltpu.CompilerParams(use_tc_tiling_on_sc=False),
        scratch_types=(
            [pltpu.VMEM((N_CHUNKS, CHUNK), jnp.int32),
             pltpu.VMEM((B_PER_W, EMBED), jnp.float32)]
            + [pltpu.VMEM((CHUNK, 128), jnp.float32) for _ in range(NBUF)]
            + [pltpu.VMEM((CHUNK + NLANE,), jnp.int32) for _ in range(NBUF)]
            + [pltpu.SemaphoreType.DMA for _ in range(2 * NBUF)]
        ),
    )
    def k(xh_hbm, hoff_hbm, table_hbm, out_hbm, idx_v, out_v, *rest):
        bufs = rest[:NBUF]
        hbufs = rest[NBUF:2 * NBUF]
        sems = rest[2 * NBUF:3 * NBUF]
        hsems = rest[3 * NBUF:]
        lanes = lax.iota(jnp.int32, NLANE)
        wid = lax.axis_index("s") * NC + lax.axis_index("c")
        ibase = wid * N_CHUNKS
        obase = wid * B_PER_W

        pltpu.sync_copy(xh_hbm.at[pl.ds(ibase, N_CHUNKS)], idx_v)

        def fire(chunk, b):
            pltpu.async_copy(table_hbm.at[idx_v.at[chunk]], bufs[b], sems[b])
            pltpu.async_copy(hoff_hbm.at[ibase + chunk],
                             hbufs[b].at[pl.ds(0, CHUNK)], hsems[b])

        def wait(chunk, b):
            pltpu.make_async_copy(
                table_hbm.at[idx_v.at[chunk]], bufs[b], sems[b]).wait()
            pltpu.make_async_copy(
                hoff_hbm.at[ibase + chunk],
                hbufs[b].at[pl.ds(0, CHUNK)], hsems[b]).wait()

        for b in range(NBUF):
            fire(b, b)

        def accum(buf, hs, accs):
            def body(s, a):
                off = hs[pl.ds(s, NLANE)][0]
                return tuple(
                    a[c] + buf[s, pl.ds(off + c * NLANE, NLANE)]
                    for c in range(NVEC)
                )
            return lax.fori_loop(0, CHUNK, body, accs)

        def outer(t, carry):
            for pair in range(NBUF // CHUNKS_PER_ITEM):
                accs = tuple(
                    jnp.zeros((NLANE,), jnp.float32) for _ in range(NVEC)
                )
                for half in range(CHUNKS_PER_ITEM):
                    b = pair * CHUNKS_PER_ITEM + half
                    chunk = t * NBUF + b
                    wait(chunk, b)
                    accs = accum(bufs[b], hbufs[b], accs)

                    @pl.when(t < N_OUTER - 1)
                    def _fire():
                        fire(chunk + NBUF, b)

                item = t * (NBUF // CHUNKS_PER_ITEM) + pair
                for c in range(NVEC):
                    out_v[item, pl.ds(c * NLANE, NLANE)] = accs[c]
            return carry

        lax.fori_loop(0, N_OUTER, outer, 0)
        pltpu.sync_copy(out_v, out_hbm.at[pl.ds(obase, B_PER_W)])

    return k(xh, hoff, table2)


VBLK = 4096
FMT_GRID = HSPLIT // VBLK
LAST_BLK = -(-VOCAB // VBLK) - 1


def _fmt_body(ta_ref, tb_ref, out_ref):
    for q in range(VBLK // 128):
        a = ta_ref[:, pl.ds(128 * q, 128)].T
        b = tb_ref[:, pl.ds(128 * q, 128)].T
        out_ref[pl.ds(128 * q, 128), :] = jnp.concatenate([a, b], axis=1)


def _tc_format(tableT):
    return pl.pallas_call(
        _fmt_body,
        grid=(FMT_GRID,),
        in_specs=[
            pl.BlockSpec((EMBED, VBLK), lambda i: (0, i)),
            pl.BlockSpec(
                (EMBED, VBLK),
                lambda i: (0, jnp.minimum(i + FMT_GRID, LAST_BLK)),
            ),
        ],
        out_specs=pl.BlockSpec((VBLK, 2 * EMBED), lambda i: (i, 0)),
        out_shape=jax.ShapeDtypeStruct((HSPLIT, 2 * EMBED), jnp.float32),
        compiler_params=pltpu.CompilerParams(
            dimension_semantics=("parallel",)),
    )(tableT, tableT)


BM = 512
NOUT_PAD = 128


def _mlp_body(s_ref, w1_ref, b1_ref, w2_ref, b2_ref, o_ref):
    h = jnp.dot(s_ref[...] * (1.0 / SEQ), w1_ref[...],
                preferred_element_type=jnp.float32)
    h = jnp.maximum(h + b1_ref[...], 0.0)
    o_ref[...] = jnp.dot(h, w2_ref[...],
                         preferred_element_type=jnp.float32) + b2_ref[...]


def kernel(x, table, W1, b1, W2, b2):
    xi = x.astype(jnp.int32)
    hi = (xi >= HSPLIT).astype(jnp.int32)
    xh = (xi - hi * HSPLIT).reshape(BATCH * SEQ // CHUNK, CHUNK)
    hoff = (hi << 6).reshape(BATCH * SEQ // CHUNK, CHUNK)
    table2 = _tc_format(table.T)
    sums = _sc_gather_sum(xh, hoff, table2)

    w2p = jnp.zeros((HIDDEN, NOUT_PAD), W2.dtype).at[:, :NOUT].set(W2)
    b2p = jnp.zeros((1, NOUT_PAD), b2.dtype).at[0, :NOUT].set(b2)
    b1r = b1.reshape(1, HIDDEN)

    out = pl.pallas_call(
        _mlp_body,
        grid=(BATCH // BM,),
        in_specs=[
            pl.BlockSpec((BM, EMBED), lambda i: (i, 0)),
            pl.BlockSpec((EMBED, HIDDEN), lambda i: (0, 0)),
            pl.BlockSpec((1, HIDDEN), lambda i: (0, 0)),
            pl.BlockSpec((HIDDEN, NOUT_PAD), lambda i: (0, 0)),
            pl.BlockSpec((1, NOUT_PAD), lambda i: (0, 0)),
        ],
        out_specs=pl.BlockSpec((BM, NOUT_PAD), lambda i: (i, 0)),
        out_shape=jax.ShapeDtypeStruct((BATCH, NOUT_PAD), jnp.float32),
    )(sums, W1, b1r, w2p, b2p)
    return out[:, :NOUT]


# speedup vs baseline: 2.6094x; 1.0955x over previous
import functools

import jax
import jax.numpy as jnp
from jax import lax
from jax.experimental import pallas as pl
from jax.experimental.pallas import tpu as pltpu
from jax.experimental.pallas import tpu_sc as plsc

VOCAB = 1000000
EMBED = 64
HIDDEN = 128
NOUT = 2
BATCH = 4096
SEQ = 200

NC = 2
NS = 16
NW = NC * NS
B_PER_W = BATCH // NW
CHUNK = 100
CHUNKS_PER_ITEM = SEQ // CHUNK
N_CHUNKS = B_PER_W * CHUNKS_PER_ITEM
NBUF = 4
N_OUTER = N_CHUNKS // NBUF
NLANE = 16
NVEC = EMBED // NLANE
HSPLIT = 512000


def _sc_gather_sum(xh, hoff, table2):
    mesh = plsc.VectorSubcoreMesh(core_axis_name="c", subcore_axis_name="s")

    @functools.partial(
        pl.kernel,
        mesh=mesh,
        out_type=jax.ShapeDtypeStruct((BATCH, EMBED), jnp.float32),
        compiler_params=pltpu.CompilerParams(use_tc_tiling_on_sc=False),
        scratch_types=(
            [pltpu.VMEM((N_CHUNKS, CHUNK), jnp.int32),
             pltpu.VMEM((B_PER_W, EMBED), jnp.float32)]
            + [pltpu.VMEM((CHUNK, 128), jnp.float32) for _ in range(NBUF)]
            + [pltpu.VMEM((CHUNK + NLANE,), jnp.int32) for _ in range(NBUF)]
            + [pltpu.SemaphoreType.DMA for _ in range(2 * NBUF)]
        ),
    )
    def k(xh_hbm, hoff_hbm, table_hbm, out_hbm, idx_v, out_v, *rest):
        bufs = rest[:NBUF]
        hbufs = rest[NBUF:2 * NBUF]
        sems = rest[2 * NBUF:3 * NBUF]
        hsems = rest[3 * NBUF:]
        lanes = lax.iota(jnp.int32, NLANE)
        wid = lax.axis_index("s") * NC + lax.axis_index("c")
        ibase = wid * N_CHUNKS
        obase = wid * B_PER_W

        pltpu.sync_copy(xh_hbm.at[pl.ds(ibase, N_CHUNKS)], idx_v)

        def fire(chunk, b):
            pltpu.async_copy(table_hbm.at[idx_v.at[chunk]], bufs[b], sems[b])
            pltpu.async_copy(hoff_hbm.at[ibase + chunk],
                             hbufs[b].at[pl.ds(0, CHUNK)], hsems[b])

        def wait(chunk, b):
            pltpu.make_async_copy(
                table_hbm.at[idx_v.at[chunk]], bufs[b], sems[b]).wait()
            pltpu.make_async_copy(
                hoff_hbm.at[ibase + chunk],
                hbufs[b].at[pl.ds(0, CHUNK)], hsems[b]).wait()

        for b in range(NBUF):
            fire(b, b)

        def accum(buf, hs, accs):
            def body(s, a):
                off = hs[pl.ds(s, NLANE)][0]
                return tuple(
                    a[c] + buf[s, pl.ds(off + c * NLANE, NLANE)]
                    for c in range(NVEC)
                )
            return lax.fori_loop(0, CHUNK, body, accs)

        def outer(t, carry):
            for pair in range(NBUF // CHUNKS_PER_ITEM):
                accs = tuple(
                    jnp.zeros((NLANE,), jnp.float32) for _ in range(NVEC)
                )
                for half in range(CHUNKS_PER_ITEM):
                    b = pair * CHUNKS_PER_ITEM + half
                    chunk = t * NBUF + b
                    wait(chunk, b)
                    accs = accum(bufs[b], hbufs[b], accs)

                    @pl.when(t < N_OUTER - 1)
                    def _fire():
                        fire(chunk + NBUF, b)

                item = t * (NBUF // CHUNKS_PER_ITEM) + pair
                for c in range(NVEC):
                    out_v[item, pl.ds(c * NLANE, NLANE)] = accs[c]
            return carry

        lax.fori_loop(0, N_OUTER, outer, 0)
        pltpu.sync_copy(out_v, out_hbm.at[pl.ds(obase, B_PER_W)])

    return k(xh, hoff, table2)


VBLK = 10240
FMT_GRID = HSPLIT // VBLK
LAST_BLK = -(-VOCAB // VBLK) - 1


def _fmt_body(ta_ref, tb_ref, out_ref):
    for q in range(VBLK // 128):
        a = ta_ref[:, pl.ds(128 * q, 128)].T
        b = tb_ref[:, pl.ds(128 * q, 128)].T
        out_ref[pl.ds(128 * q, 128), :] = jnp.concatenate([a, b], axis=1)


def _tc_format(tableT):
    return pl.pallas_call(
        _fmt_body,
        grid=(FMT_GRID,),
        in_specs=[
            pl.BlockSpec((EMBED, VBLK), lambda i: (0, i)),
            pl.BlockSpec(
                (EMBED, VBLK),
                lambda i: (0, jnp.minimum(i + FMT_GRID, LAST_BLK)),
            ),
        ],
        out_specs=pl.BlockSpec((VBLK, 2 * EMBED), lambda i: (i, 0)),
        out_shape=jax.ShapeDtypeStruct((HSPLIT, 2 * EMBED), jnp.float32),
        compiler_params=pltpu.CompilerParams(
            dimension_semantics=("parallel",)),
    )(tableT, tableT)


BM = 512
NOUT_PAD = 128


def _mlp_body(s_ref, w1_ref, b1_ref, w2_ref, b2_ref, o_ref):
    h = jnp.dot(s_ref[...] * (1.0 / SEQ), w1_ref[...],
                preferred_element_type=jnp.float32)
    h = jnp.maximum(h + b1_ref[...], 0.0)
    o_ref[...] = jnp.dot(h, w2_ref[...],
                         preferred_element_type=jnp.float32) + b2_ref[...]


def kernel(x, table, W1, b1, W2, b2):
    xi = x.astype(jnp.int32)
    hi = (xi >= HSPLIT).astype(jnp.int32)
    xh = (xi - hi * HSPLIT).reshape(BATCH * SEQ // CHUNK, CHUNK)
    hoff = (hi << 6).reshape(BATCH * SEQ // CHUNK, CHUNK)
    table2 = _tc_format(table.T)
    sums = _sc_gather_sum(xh, hoff, table2)

    w2p = jnp.zeros((HIDDEN, NOUT_PAD), W2.dtype).at[:, :NOUT].set(W2)
    b2p = jnp.zeros((1, NOUT_PAD), b2.dtype).at[0, :NOUT].set(b2)
    b1r = b1.reshape(1, HIDDEN)

    out = pl.pallas_call(
        _mlp_body,
        grid=(BATCH // BM,),
        in_specs=[
            pl.BlockSpec((BM, EMBED), lambda i: (i, 0)),
            pl.BlockSpec((EMBED, HIDDEN), lambda i: (0, 0)),
            pl.BlockSpec((1, HIDDEN), lambda i: (0, 0)),
            pl.BlockSpec((HIDDEN, NOUT_PAD), lambda i: (0, 0)),
            pl.BlockSpec((1, NOUT_PAD), lambda i: (0, 0)),
        ],
        out_specs=pl.BlockSpec((BM, NOUT_PAD), lambda i: (i, 0)),
        out_shape=jax.ShapeDtypeStruct((BATCH, NOUT_PAD), jnp.float32),
    )(sums, W1, b1r, w2p, b2p)
    return out[:, :NOUT]


# speedup vs baseline: 2.6549x; 1.0174x over previous
import functools

import jax
import jax.numpy as jnp
from jax import lax
from jax.experimental import pallas as pl
from jax.experimental.pallas import tpu as pltpu
from jax.experimental.pallas import tpu_sc as plsc

VOCAB = 1000000
EMBED = 64
HIDDEN = 128
NOUT = 2
BATCH = 4096
SEQ = 200

NC = 2
NS = 16
NW = NC * NS
B_PER_W = BATCH // NW
CHUNK = 100
CHUNKS_PER_ITEM = SEQ // CHUNK
N_CHUNKS = B_PER_W * CHUNKS_PER_ITEM
NBUF = 4
N_OUTER = N_CHUNKS // NBUF
NLANE = 16
NVEC = EMBED // NLANE
HSPLIT = 512000


def _sc_gather_sum(xh, hoff, table2):
    mesh = plsc.VectorSubcoreMesh(core_axis_name="c", subcore_axis_name="s")

    @functools.partial(
        pl.kernel,
        mesh=mesh,
        out_type=jax.ShapeDtypeStruct((BATCH, EMBED), jnp.float32),
        compiler_params=pltpu.CompilerParams(use_tc_tiling_on_sc=False),
        scratch_types=(
            [pltpu.VMEM((N_CHUNKS, CHUNK), jnp.int32),
             pltpu.VMEM((B_PER_W, EMBED), jnp.float32)]
            + [pltpu.VMEM((CHUNK, 128), jnp.float32) for _ in range(NBUF)]
            + [pltpu.VMEM((CHUNK + NLANE,), jnp.int32) for _ in range(NBUF)]
            + [pltpu.SemaphoreType.DMA for _ in range(2 * NBUF)]
        ),
    )
    def k(xh_hbm, hoff_hbm, table_hbm, out_hbm, idx_v, out_v, *rest):
        bufs = rest[:NBUF]
        hbufs = rest[NBUF:2 * NBUF]
        sems = rest[2 * NBUF:3 * NBUF]
        hsems = rest[3 * NBUF:]
        lanes = lax.iota(jnp.int32, NLANE)
        wid = lax.axis_index("s") * NC + lax.axis_index("c")
        ibase = wid * N_CHUNKS
        obase = wid * B_PER_W

        pltpu.sync_copy(xh_hbm.at[pl.ds(ibase, N_CHUNKS)], idx_v)

        def fire(chunk, b):
            pltpu.async_copy(table_hbm.at[idx_v.at[chunk]], bufs[b], sems[b])
            pltpu.async_copy(hoff_hbm.at[ibase + chunk],
                             hbufs[b].at[pl.ds(0, CHUNK)], hsems[b])

        def wait(chunk, b):
            pltpu.make_async_copy(
                table_hbm.at[idx_v.at[chunk]], bufs[b], sems[b]).wait()
            pltpu.make_async_copy(
                hoff_hbm.at[ibase + chunk],
                hbufs[b].at[pl.ds(0, CHUNK)], hsems[b]).wait()

        for b in range(NBUF):
            fire(b, b)

        def accum(buf, hs, accs):
            def body(s, a):
                off = hs[pl.ds(s, NLANE)][0]
                return tuple(
                    a[c] + buf[s, pl.ds(off + c * NLANE, NLANE)]
                    for c in range(NVEC)
                )
            return lax.fori_loop(0, CHUNK, body, accs)

        def outer(t, carry):
            for pair in range(NBUF // CHUNKS_PER_ITEM):
                accs = tuple(
                    jnp.zeros((NLANE,), jnp.float32) for _ in range(NVEC)
                )
                for half in range(CHUNKS_PER_ITEM):
                    b = pair * CHUNKS_PER_ITEM + half
                    chunk = t * NBUF + b
                    wait(chunk, b)
                    accs = accum(bufs[b], hbufs[b], accs)

                    @pl.when(t < N_OUTER - 1)
                    def _fire():
                        fire(chunk + NBUF, b)

                item = t * (NBUF // CHUNKS_PER_ITEM) + pair
                for c in range(NVEC):
                    out_v[item, pl.ds(c * NLANE, NLANE)] = accs[c]
            return carry

        lax.fori_loop(0, N_OUTER, outer, 0)
        pltpu.sync_copy(out_v, out_hbm.at[pl.ds(obase, B_PER_W)])

    return k(xh, hoff, table2)


VBLK = 16000
FMT_GRID = HSPLIT // VBLK
LAST_BLK = -(-VOCAB // VBLK) - 1


def _fmt_body(ta_ref, tb_ref, out_ref):
    for q in range(VBLK // 128):
        a = ta_ref[:, pl.ds(128 * q, 128)].T
        b = tb_ref[:, pl.ds(128 * q, 128)].T
        out_ref[pl.ds(128 * q, 128), :] = jnp.concatenate([a, b], axis=1)


def _tc_format(tableT):
    return pl.pallas_call(
        _fmt_body,
        grid=(FMT_GRID,),
        in_specs=[
            pl.BlockSpec((EMBED, VBLK), lambda i: (0, i)),
            pl.BlockSpec(
                (EMBED, VBLK),
                lambda i: (0, jnp.minimum(i + FMT_GRID, LAST_BLK)),
            ),
        ],
        out_specs=pl.BlockSpec((VBLK, 2 * EMBED), lambda i: (i, 0)),
        out_shape=jax.ShapeDtypeStruct((HSPLIT, 2 * EMBED), jnp.float32),
        compiler_params=pltpu.CompilerParams(
            dimension_semantics=("parallel",)),
    )(tableT, tableT)


BM = 512
NOUT_PAD = 128


def _mlp_body(s_ref, w1_ref, b1_ref, w2_ref, b2_ref, o_ref):
    h = jnp.dot(s_ref[...] * (1.0 / SEQ), w1_ref[...],
                preferred_element_type=jnp.float32)
    h = jnp.maximum(h + b1_ref[...], 0.0)
    o_ref[...] = jnp.dot(h, w2_ref[...],
                         preferred_element_type=jnp.float32) + b2_ref[...]


def kernel(x, table, W1, b1, W2, b2):
    xi = x.astype(jnp.int32)
    hi = (xi >= HSPLIT).astype(jnp.int32)
    xh = (xi - hi * HSPLIT).reshape(BATCH * SEQ // CHUNK, CHUNK)
    hoff = (hi << 6).reshape(BATCH * SEQ // CHUNK, CHUNK)
    table2 = _tc_format(table.T)
    sums = _sc_gather_sum(xh, hoff, table2)

    w2p = jnp.zeros((HIDDEN, NOUT_PAD), W2.dtype).at[:, :NOUT].set(W2)
    b2p = jnp.zeros((1, NOUT_PAD), b2.dtype).at[0, :NOUT].set(b2)
    b1r = b1.reshape(1, HIDDEN)

    out = pl.pallas_call(
        _mlp_body,
        grid=(BATCH // BM,),
        in_specs=[
            pl.BlockSpec((BM, EMBED), lambda i: (i, 0)),
            pl.BlockSpec((EMBED, HIDDEN), lambda i: (0, 0)),
            pl.BlockSpec((1, HIDDEN), lambda i: (0, 0)),
            pl.BlockSpec((HIDDEN, NOUT_PAD), lambda i: (0, 0)),
            pl.BlockSpec((1, NOUT_PAD), lambda i: (0, 0)),
        ],
        out_specs=pl.BlockSpec((BM, NOUT_PAD), lambda i: (i, 0)),
        out_shape=jax.ShapeDtypeStruct((BATCH, NOUT_PAD), jnp.float32),
    )(sums, W1, b1r, w2p, b2p)
    return out[:, :NOUT]
